# baseline (device time: 36238 ns/iter reference)
import jax
import jax.numpy as jnp
from jax import lax
from jax.experimental import pallas as pl
from jax.experimental.pallas import tpu as pltpu

N_DEV = 16
B = 256
D = 256
BLK = B // N_DEV

GROUPS = [(1, 5), (5, 9), (9, 13), (13, 16)]


def kernel(x, Win0, Wout0, Win1, Wout1, Win2, Wout2):
    def body(x_ref, win0_ref, wout0_ref, win1_ref, wout1_ref, win2_ref,
             wout2_ref, out_ref, p_ref, rs_buf, xg_buf,
             rs_send_sems, ag_send_sems, rs_sems, ag_sems):
        my = lax.axis_index("i")

        barrier = pltpu.get_barrier_semaphore()
        for k in range(1, N_DEV):
            pl.semaphore_signal(
                barrier, inc=1,
                device_id=((my + k) % N_DEV,),
                device_id_type=pl.DeviceIdType.MESH,
            )
        pl.semaphore_wait(barrier, N_DEV - 1)

        wins = [win0_ref, win1_ref, win2_ref]
        wouts = [wout0_ref, wout1_ref, wout2_ref]

        def mlp(xv, w_in, w_out):
            h = jnp.dot(xv.astype(jnp.bfloat16), w_in,
                        preferred_element_type=jnp.float32)
            h = jnp.maximum(h, 0.0)
            return jnp.dot(h.astype(jnp.bfloat16), w_out,
                           preferred_element_type=jnp.float32)

        w_in = win0_ref[...].astype(jnp.bfloat16)
        w_out = wout0_ref[...].astype(jnp.bfloat16)
        p_ref[...] = mlp(x_ref[...], w_in, w_out).astype(jnp.bfloat16)

        rs = []
        for k in range(1, N_DEV):
            dst = (my + k) % N_DEV
            r = pltpu.make_async_remote_copy(
                src_ref=p_ref.at[pl.ds(dst * BLK, BLK), :],
                dst_ref=rs_buf.at[k],
                send_sem=rs_send_sems.at[k],
                recv_sem=rs_sems.at[k],
                device_id=(dst,),
                device_id_type=pl.DeviceIdType.MESH,
            )
            r.start()
            rs.append(r)
        rs_buf[0, :, :] = p_ref[pl.ds(my * BLK, BLK), :]
        for r in rs:
            r.wait_recv()
        for r in rs:
            r.wait_send()
        acc = jnp.sum(rs_buf[...].astype(jnp.float32), axis=0)

        for layer in (1, 2):
            w_in = wins[layer][...].astype(jnp.bfloat16)
            w_out = wouts[layer][...].astype(jnp.bfloat16)

            xg_buf[0, :, :] = acc.astype(jnp.bfloat16)
            ag = []
            for k in range(1, N_DEV):
                dst = (my + k) % N_DEV
                r = pltpu.make_async_remote_copy(
                    src_ref=xg_buf.at[0],
                    dst_ref=xg_buf.at[k],
                    send_sem=ag_send_sems.at[k],
                    recv_sem=ag_sems.at[k],
                    device_id=(dst,),
                    device_id_type=pl.DeviceIdType.MESH,
                )
                r.start()
                ag.append(r)

            rs = []
            rs_buf[0, :, :] = mlp(acc, w_in, w_out).astype(jnp.bfloat16)

            for lo, hi in GROUPS:
                for k in range(lo, hi):
                    ag[k - 1].wait_recv()
                xgrp = xg_buf[lo:hi].reshape((hi - lo) * BLK, D)
                pgrp = mlp(xgrp, w_in, w_out).astype(jnp.bfloat16)
                p_ref[pl.ds(lo * BLK, (hi - lo) * BLK), :] = pgrp
                for k in range(lo, hi):
                    owner = (my - k) % N_DEV
                    r = pltpu.make_async_remote_copy(
                        src_ref=p_ref.at[pl.ds(k * BLK, BLK), :],
                        dst_ref=rs_buf.at[N_DEV - k],
                        send_sem=rs_send_sems.at[N_DEV - k],
                        recv_sem=rs_sems.at[N_DEV - k],
                        device_id=(owner,),
                        device_id_type=pl.DeviceIdType.MESH,
                    )
                    r.start()
                    rs.append(r)

            for r in rs:
                r.wait_recv()
            for r in rs:
                r.wait_send()
            for r in ag:
                r.wait_send()
            acc = jnp.sum(rs_buf[...].astype(jnp.float32), axis=0)

        out_ref[...] = acc

    return pl.pallas_call(
        body,
        out_shape=jax.ShapeDtypeStruct((BLK, D), jnp.float32),
        in_specs=[pl.BlockSpec(memory_space=pltpu.VMEM)] * 7,
        out_specs=pl.BlockSpec(memory_space=pltpu.VMEM),
        scratch_shapes=[
            pltpu.VMEM((B, D), jnp.bfloat16),
            pltpu.VMEM((N_DEV, BLK, D), jnp.bfloat16),
            pltpu.VMEM((N_DEV, BLK, D), jnp.bfloat16),
            pltpu.SemaphoreType.DMA((N_DEV,)),
            pltpu.SemaphoreType.DMA((N_DEV,)),
            pltpu.SemaphoreType.DMA((N_DEV,)),
            pltpu.SemaphoreType.DMA((N_DEV,)),
        ],
        compiler_params=pltpu.CompilerParams(collective_id=0),
    )(x, Win0, Wout0, Win1, Wout1, Win2, Wout2)


# device time: 35728 ns/iter; 1.0143x vs baseline; 1.0143x over previous
import jax
import jax.numpy as jnp
from jax import lax
from jax.experimental import pallas as pl
from jax.experimental.pallas import tpu as pltpu

N_DEV = 16
B = 256
D = 256
BLK = B // N_DEV


def kernel(x, Win0, Wout0, Win1, Wout1, Win2, Wout2):
    def body(x_ref, win0_ref, wout0_ref, win1_ref, wout1_ref, win2_ref,
             wout2_ref, out_ref, p_ref, rs_buf, x_buf,
             rs_send_sems, ag_send_sems, rs_sems, ag_sems):
        my = lax.axis_index("i")

        barrier = pltpu.get_barrier_semaphore()
        for k in range(1, N_DEV):
            pl.semaphore_signal(
                barrier, inc=1,
                device_id=((my + k) % N_DEV,),
                device_id_type=pl.DeviceIdType.MESH,
            )

        wins = [win0_ref, win1_ref, win2_ref]
        wouts = [wout0_ref, wout1_ref, wout2_ref]

        def mlp(xv, w_in, w_out):
            h = jnp.dot(xv.astype(jnp.bfloat16), w_in,
                        preferred_element_type=jnp.float32)
            h = jnp.maximum(h, 0.0)
            return jnp.dot(h.astype(jnp.bfloat16), w_out,
                           preferred_element_type=jnp.float32)

        w_in = win0_ref[...].astype(jnp.bfloat16)
        w_out = wout0_ref[...].astype(jnp.bfloat16)
        p_ref[...] = mlp(x_ref[...], w_in, w_out).astype(jnp.bfloat16)
        rs_buf[0, :, :] = p_ref[pl.ds(my * BLK, BLK), :]

        pl.semaphore_wait(barrier, N_DEV - 1)

        acc = None
        for layer in range(3):
            rs = []
            for k in range(1, N_DEV):
                dst = (my + k) % N_DEV
                r = pltpu.make_async_remote_copy(
                    src_ref=p_ref.at[pl.ds(dst * BLK, BLK), :],
                    dst_ref=rs_buf.at[k],
                    send_sem=rs_send_sems.at[k],
                    recv_sem=rs_sems.at[k],
                    device_id=(dst,),
                    device_id_type=pl.DeviceIdType.MESH,
                )
                r.start()
                rs.append(r)
            if layer < 2:
                w_in = wins[layer + 1][...].astype(jnp.bfloat16)
                w_out = wouts[layer + 1][...].astype(jnp.bfloat16)
            for r in rs:
                r.wait_recv()
            for r in rs:
                r.wait_send()
            acc = jnp.sum(rs_buf[...].astype(jnp.float32), axis=0)

            if layer == 2:
                break

            x_buf[pl.ds(my * BLK, BLK), :] = acc.astype(jnp.bfloat16)
            ag = []
            for k in range(1, N_DEV):
                dst = (my + k) % N_DEV
                r = pltpu.make_async_remote_copy(
                    src_ref=x_buf.at[pl.ds(my * BLK, BLK), :],
                    dst_ref=x_buf.at[pl.ds(my * BLK, BLK), :],
                    send_sem=ag_send_sems.at[k],
                    recv_sem=ag_sems.at[k],
                    device_id=(dst,),
                    device_id_type=pl.DeviceIdType.MESH,
                )
                r.start()
                ag.append(r)
            for r in ag:
                r.wait_recv()
            for r in ag:
                r.wait_send()

            p_ref[...] = mlp(x_buf[...], w_in, w_out).astype(jnp.bfloat16)
            rs_buf[0, :, :] = p_ref[pl.ds(my * BLK, BLK), :]

        out_ref[...] = acc

    return pl.pallas_call(
        body,
        out_shape=jax.ShapeDtypeStruct((BLK, D), jnp.float32),
        in_specs=[pl.BlockSpec(memory_space=pltpu.VMEM)] * 7,
        out_specs=pl.BlockSpec(memory_space=pltpu.VMEM),
        scratch_shapes=[
            pltpu.VMEM((B, D), jnp.bfloat16),
            pltpu.VMEM((N_DEV, BLK, D), jnp.bfloat16),
            pltpu.VMEM((B, D), jnp.bfloat16),
            pltpu.SemaphoreType.DMA((N_DEV,)),
            pltpu.SemaphoreType.DMA((N_DEV,)),
            pltpu.SemaphoreType.DMA((N_DEV,)),
            pltpu.SemaphoreType.DMA((N_DEV,)),
        ],
        compiler_params=pltpu.CompilerParams(collective_id=0),
    )(x, Win0, Wout0, Win1, Wout1, Win2, Wout2)


# device time: 35556 ns/iter; 1.0192x vs baseline; 1.0048x over previous
import jax
import jax.numpy as jnp
from jax import lax
from jax.experimental import pallas as pl
from jax.experimental.pallas import tpu as pltpu

N_DEV = 16
B = 256
D = 256
BLK = B // N_DEV


def kernel(x, Win0, Wout0, Win1, Wout1, Win2, Wout2):
    def body(x_ref, win0_ref, wout0_ref, win1_ref, wout1_ref, win2_ref,
             wout2_ref, out_ref, p_ref, rs_buf, x_buf,
             rs_send_sems, ag_send_sems, rs_sems, ag_sems, entry_sems):
        my = lax.axis_index("i")

        barrier = pltpu.get_barrier_semaphore()
        pl.semaphore_signal(barrier, inc=1)
        pl.semaphore_wait(barrier, 1)

        for j in range(1, N_DEV):
            pl.semaphore_signal(
                entry_sems.at[j], inc=1,
                device_id=((my + j) % N_DEV,),
                device_id_type=pl.DeviceIdType.MESH,
            )

        wins = [win0_ref, win1_ref, win2_ref]
        wouts = [wout0_ref, wout1_ref, wout2_ref]

        def mlp(xv, w_in, w_out):
            h = jnp.dot(xv.astype(jnp.bfloat16), w_in,
                        preferred_element_type=jnp.float32)
            h = jnp.maximum(h, 0.0)
            return jnp.dot(h.astype(jnp.bfloat16), w_out,
                           preferred_element_type=jnp.float32)

        w_in = win0_ref[...].astype(jnp.bfloat16)
        w_out = wout0_ref[...].astype(jnp.bfloat16)
        p_ref[...] = mlp(x_ref[...], w_in, w_out).astype(jnp.bfloat16)
        rs_buf[0, 0, :, :] = p_ref[pl.ds(my * BLK, BLK), :]

        acc = None
        for layer in range(3):
            rs = []
            for k in range(1, N_DEV):
                dst = (my + k) % N_DEV
                if layer == 0:
                    pl.semaphore_wait(entry_sems.at[N_DEV - k], 1)
                r = pltpu.make_async_remote_copy(
                    src_ref=p_ref.at[pl.ds(dst * BLK, BLK), :],
                    dst_ref=rs_buf.at[layer, k],
                    send_sem=rs_send_sems.at[k],
                    recv_sem=rs_sems.at[layer, k],
                    device_id=(dst,),
                    device_id_type=pl.DeviceIdType.MESH,
                )
                r.start()
                rs.append(r)
            if layer < 2:
                w_in = wins[layer + 1][...].astype(jnp.bfloat16)
                w_out = wouts[layer + 1][...].astype(jnp.bfloat16)
            for r in rs:
                r.wait_recv()
            for r in rs:
                r.wait_send()
            acc = jnp.sum(rs_buf[layer].astype(jnp.float32), axis=0)

            if layer == 2:
                break

            x_buf[layer, pl.ds(my * BLK, BLK), :] = acc.astype(jnp.bfloat16)
            ag = []
            for k in range(1, N_DEV):
                dst = (my + k) % N_DEV
                r = pltpu.make_async_remote_copy(
                    src_ref=x_buf.at[layer, pl.ds(my * BLK, BLK), :],
                    dst_ref=x_buf.at[layer, pl.ds(my * BLK, BLK), :],
                    send_sem=ag_send_sems.at[k],
                    recv_sem=ag_sems.at[layer, k],
                    device_id=(dst,),
                    device_id_type=pl.DeviceIdType.MESH,
                )
                r.start()
                ag.append(r)
            for r in ag:
                r.wait_recv()
            for r in ag:
                r.wait_send()

            p_ref[...] = mlp(x_buf[layer], w_in, w_out).astype(jnp.bfloat16)
            rs_buf[layer + 1, 0, :, :] = p_ref[pl.ds(my * BLK, BLK), :]

        out_ref[...] = acc

    return pl.pallas_call(
        body,
        out_shape=jax.ShapeDtypeStruct((BLK, D), jnp.float32),
        in_specs=[pl.BlockSpec(memory_space=pltpu.VMEM)] * 7,
        out_specs=pl.BlockSpec(memory_space=pltpu.VMEM),
        scratch_shapes=[
            pltpu.VMEM((B, D), jnp.bfloat16),
            pltpu.VMEM((3, N_DEV, BLK, D), jnp.bfloat16),
            pltpu.VMEM((2, B, D), jnp.bfloat16),
            pltpu.SemaphoreType.DMA((N_DEV,)),
            pltpu.SemaphoreType.DMA((N_DEV,)),
            pltpu.SemaphoreType.DMA((3, N_DEV)),
            pltpu.SemaphoreType.DMA((2, N_DEV)),
            pltpu.SemaphoreType.REGULAR((N_DEV,)),
        ],
        compiler_params=pltpu.CompilerParams(collective_id=0),
    )(x, Win0, Wout0, Win1, Wout1, Win2, Wout2)


# device time: 34749 ns/iter; 1.0429x vs baseline; 1.0232x over previous
import jax
import jax.numpy as jnp
from jax import lax
from jax.experimental import pallas as pl
from jax.experimental.pallas import tpu as pltpu

N_DEV = 16
B = 256
D = 256
BLK = B // N_DEV


def kernel(x, Win0, Wout0, Win1, Wout1, Win2, Wout2):
    x, Win0, Wout0, Win1, Wout1, Win2, Wout2 = (
        a.astype(jnp.bfloat16)
        for a in (x, Win0, Wout0, Win1, Wout1, Win2, Wout2)
    )

    def body(x_ref, win0_ref, wout0_ref, win1_ref, wout1_ref, win2_ref,
             wout2_ref, out_ref, p_ref, rs_buf, x_buf,
             rs_send_sems, ag_send_sems, rs_sems, ag_sems, entry_sems):
        my = lax.axis_index("i")

        barrier = pltpu.get_barrier_semaphore()
        pl.semaphore_signal(barrier, inc=1)
        pl.semaphore_wait(barrier, 1)

        for j in range(1, N_DEV):
            pl.semaphore_signal(
                entry_sems.at[j], inc=1,
                device_id=((my + j) % N_DEV,),
                device_id_type=pl.DeviceIdType.MESH,
            )

        wins = [win0_ref, win1_ref, win2_ref]
        wouts = [wout0_ref, wout1_ref, wout2_ref]

        def mlp(xv, w_in, w_out):
            h = jnp.dot(xv.astype(jnp.bfloat16), w_in,
                        preferred_element_type=jnp.float32)
            h = jnp.maximum(h, 0.0)
            return jnp.dot(h.astype(jnp.bfloat16), w_out,
                           preferred_element_type=jnp.float32)

        w_in = win0_ref[...]
        w_out = wout0_ref[...]
        p_ref[...] = mlp(x_ref[...], w_in, w_out).astype(jnp.bfloat16)
        rs_buf[0, 0, :, :] = p_ref[pl.ds(my * BLK, BLK), :]

        acc = None
        for layer in range(3):
            rs = []
            for k in range(1, N_DEV):
                dst = (my + k) % N_DEV
                if layer == 0:
                    pl.semaphore_wait(entry_sems.at[N_DEV - k], 1)
                r = pltpu.make_async_remote_copy(
                    src_ref=p_ref.at[pl.ds(dst * BLK, BLK), :],
                    dst_ref=rs_buf.at[layer, k],
                    send_sem=rs_send_sems.at[k],
                    recv_sem=rs_sems.at[layer, k],
                    device_id=(dst,),
                    device_id_type=pl.DeviceIdType.MESH,
                )
                r.start()
                rs.append(r)
            if layer < 2:
                w_in = wins[layer + 1][...]
                w_out = wouts[layer + 1][...]
            for r in rs:
                r.wait_recv()
            for r in rs:
                r.wait_send()
            acc = jnp.sum(rs_buf[layer].astype(jnp.float32), axis=0)

            if layer == 2:
                break

            x_buf[layer, pl.ds(my * BLK, BLK), :] = acc.astype(jnp.bfloat16)
            ag = []
            for k in range(1, N_DEV):
                dst = (my + k) % N_DEV
                r = pltpu.make_async_remote_copy(
                    src_ref=x_buf.at[layer, pl.ds(my * BLK, BLK), :],
                    dst_ref=x_buf.at[layer, pl.ds(my * BLK, BLK), :],
                    send_sem=ag_send_sems.at[k],
                    recv_sem=ag_sems.at[layer, k],
                    device_id=(dst,),
                    device_id_type=pl.DeviceIdType.MESH,
                )
                r.start()
                ag.append(r)
            for r in ag:
                r.wait_recv()
            for r in ag:
                r.wait_send()

            p_ref[...] = mlp(x_buf[layer], w_in, w_out).astype(jnp.bfloat16)
            rs_buf[layer + 1, 0, :, :] = p_ref[pl.ds(my * BLK, BLK), :]

        out_ref[...] = acc

    return pl.pallas_call(
        body,
        out_shape=jax.ShapeDtypeStruct((BLK, D), jnp.float32),
        in_specs=[pl.BlockSpec(memory_space=pltpu.VMEM)] * 7,
        out_specs=pl.BlockSpec(memory_space=pltpu.VMEM),
        scratch_shapes=[
            pltpu.VMEM((B, D), jnp.bfloat16),
            pltpu.VMEM((3, N_DEV, BLK, D), jnp.bfloat16),
            pltpu.VMEM((2, B, D), jnp.bfloat16),
            pltpu.SemaphoreType.DMA((N_DEV,)),
            pltpu.SemaphoreType.DMA((N_DEV,)),
            pltpu.SemaphoreType.DMA((3, N_DEV)),
            pltpu.SemaphoreType.DMA((2, N_DEV)),
            pltpu.SemaphoreType.REGULAR((N_DEV,)),
        ],
        compiler_params=pltpu.CompilerParams(collective_id=0),
    )(x, Win0, Wout0, Win1, Wout1, Win2, Wout2)
